# baseline (speedup 1.0000x reference)
import jax
import jax.numpy as jnp
from jax.experimental import pallas as pl
from jax.experimental.pallas import tpu as pltpu
from jax.experimental.pallas import tpu_sc as plsc

_ROWS = 16
_LANES = 16


def kernel(x, pos_table):
    B, T, D = x.shape
    x2 = x.reshape(B * T, D)
    nb_per_batch = T // _ROWS
    mesh = plsc.VectorSubcoreMesh(core_axis_name="c", subcore_axis_name="s")

    @pl.kernel(out_type=jax.ShapeDtypeStruct((B * T, D), x.dtype), mesh=mesh)
    def sc_kernel(x_hbm, pos_hbm, o_hbm):
        def body(x_vmem, pos_vmem, o_vmem):
            @pl.loop(0, _ROWS)
            def _(r):
                @plsc.parallel_loop(0, D, step=_LANES, unroll=8)
                def _(c):
                    slc = (pl.ds(r, 1), pl.ds(c, _LANES))
                    o_vmem.at[*slc][...] = (
                        x_vmem.at[*slc][...] + pos_vmem.at[*slc][...]
                    )

        pltpu.emit_pipeline(
            body,
            grid=(B * T // _ROWS,),
            in_specs=[
                pl.BlockSpec((_ROWS, D), lambda i: (i, 0)),
                pl.BlockSpec((_ROWS, D), lambda i: (i % nb_per_batch, 0)),
            ],
            out_specs=[pl.BlockSpec((_ROWS, D), lambda i: (i, 0))],
            core_axis_name=("c", "s"),
            dimension_semantics=(pltpu.PARALLEL,),
        )(x_hbm, pos_hbm, o_hbm)

    return sc_kernel(x2, pos_table).reshape(B, T, D)


# speedup vs baseline: 1.1691x; 1.1691x over previous
import jax
import jax.numpy as jnp
from jax.experimental import pallas as pl
from jax.experimental.pallas import tpu as pltpu
from jax.experimental.pallas import tpu_sc as plsc

_ROWS = 8
_LANES = 16


def kernel(x, pos_table):
    B, T, D = x.shape
    x2 = x.reshape(B * T, D)
    nb = T // _ROWS
    mesh = plsc.VectorSubcoreMesh(core_axis_name="c", subcore_axis_name="s")

    @pl.kernel(out_type=jax.ShapeDtypeStruct((B * T, D), x.dtype), mesh=mesh)
    def sc_kernel(x_hbm, pos_hbm, o_hbm):
        def body(xa, xb, pos, oa, ob):
            for xv, ov in ((xa, oa), (xb, ob)):
                @pl.loop(0, _ROWS)
                def _(r, xv=xv, ov=ov):
                    @plsc.parallel_loop(0, D, step=_LANES, unroll=8)
                    def _(c):
                        slc = (pl.ds(r, 1), pl.ds(c, _LANES))
                        ov.at[*slc][...] = xv.at[*slc][...] + pos.at[*slc][...]

        def x_spec(b):
            return pl.BlockSpec(
                (_ROWS, D), lambda h, i, b=b: ((2 * h + b) * nb + i, 0))

        pltpu.emit_pipeline(
            body,
            grid=(2, nb),
            in_specs=[x_spec(0), x_spec(1),
                      pl.BlockSpec((_ROWS, D), lambda h, i: (i, 0))],
            out_specs=[x_spec(0), x_spec(1)],
            core_axis_name=("c", "s"),
            dimension_semantics=(pltpu.PARALLEL, pltpu.PARALLEL),
        )(x_hbm, x_hbm, pos_hbm, o_hbm, o_hbm)

    return sc_kernel(x2, pos_table).reshape(B, T, D)


# speedup vs baseline: 1.2073x; 1.0327x over previous
import jax
import jax.numpy as jnp
from jax.experimental import pallas as pl
from jax.experimental.pallas import tpu as pltpu
from jax.experimental.pallas import tpu_sc as plsc

_ROWS = 4
_LANES = 16


def kernel(x, pos_table):
    B, T, D = x.shape
    x2 = x.reshape(B * T, D)
    nb = T // _ROWS
    mesh = plsc.VectorSubcoreMesh(core_axis_name="c", subcore_axis_name="s")

    @pl.kernel(out_type=jax.ShapeDtypeStruct((B * T, D), x.dtype), mesh=mesh)
    def sc_kernel(x_hbm, pos_hbm, o_hbm):
        def body(x0, x1, x2_, x3, pos, o0, o1, o2, o3):
            for xv, ov in ((x0, o0), (x1, o1), (x2_, o2), (x3, o3)):
                @pl.loop(0, _ROWS)
                def _(r, xv=xv, ov=ov):
                    @plsc.parallel_loop(0, D, step=_LANES, unroll=8)
                    def _(c):
                        slc = (pl.ds(r, 1), pl.ds(c, _LANES))
                        ov.at[*slc][...] = xv.at[*slc][...] + pos.at[*slc][...]

        def x_spec(b):
            return pl.BlockSpec(
                (_ROWS, D), lambda i, b=b: (b * nb + i, 0))

        pltpu.emit_pipeline(
            body,
            grid=(nb,),
            in_specs=[x_spec(0), x_spec(1), x_spec(2), x_spec(3),
                      pl.BlockSpec((_ROWS, D), lambda i: (i, 0))],
            out_specs=[x_spec(0), x_spec(1), x_spec(2), x_spec(3)],
            core_axis_name=("c", "s"),
            dimension_semantics=(pltpu.PARALLEL,),
        )(x_hbm, x_hbm, x_hbm, x_hbm, pos_hbm, o_hbm, o_hbm, o_hbm, o_hbm)

    return sc_kernel(x2, pos_table).reshape(B, T, D)


# speedup vs baseline: 1.2380x; 1.0254x over previous
import jax
import jax.numpy as jnp
from jax.experimental import pallas as pl
from jax.experimental.pallas import tpu as pltpu
from jax.experimental.pallas import tpu_sc as plsc

_ROWS = 4
_LANES = 16


def kernel(x, pos_table):
    B, T, D = x.shape
    nb = T // _ROWS
    mesh = plsc.VectorSubcoreMesh(core_axis_name="c", subcore_axis_name="s")

    @pl.kernel(out_type=jax.ShapeDtypeStruct(x.shape, x.dtype), mesh=mesh)
    def sc_kernel(x_hbm, pos_hbm, o_hbm):
        def body(xv, pos, ov):
            for b in range(B):
                xb = xv.at[b]
                ob = ov.at[b]

                @pl.loop(0, _ROWS)
                def _(r, xb=xb, ob=ob):
                    @plsc.parallel_loop(0, D, step=_LANES, unroll=8)
                    def _(c):
                        slc = (pl.ds(r, 1), pl.ds(c, _LANES))
                        ob.at[*slc][...] = xb.at[*slc][...] + pos.at[*slc][...]

        spec3 = pl.BlockSpec((B, _ROWS, D), lambda i: (0, i, 0))
        pltpu.emit_pipeline(
            body,
            grid=(nb,),
            in_specs=[spec3, pl.BlockSpec((_ROWS, D), lambda i: (i, 0))],
            out_specs=[spec3],
            core_axis_name=("c", "s"),
            dimension_semantics=(pltpu.PARALLEL,),
        )(x_hbm, pos_hbm, o_hbm)

    return sc_kernel(x, pos_table)
